# Initial kernel scaffold; baseline (speedup 1.0000x reference)
import jax
import jax.numpy as jnp
from jax.experimental import pallas as pl


def kernel(hidden_states, Wq, Wk, Wv, Wg, Wo):
    raise NotImplementedError("write your pallas kernel here")


# speedup vs baseline: 1.2060x; 1.2060x over previous
import functools

import jax
import jax.numpy as jnp
from jax.experimental import pallas as pl

HIDDEN = 2048
H = 16
HKV = 4
G = H // HKV
D = 64
BS = 64
SBLK = 16
WIN = 512
THETA = 10000.0
NEG = -1e9

RB = 256
TQ = 256


def _rope2d(x, cosb, sinb):
    j = jax.lax.broadcasted_iota(jnp.int32, x.shape, 1) % 64
    lo = jnp.roll(x, -32, axis=1)
    hi = jnp.roll(x, 32, axis=1)
    partner = jnp.where(j < 32, lo, hi)
    return x * cosb + partner * sinb


def _k1_body(hs_ref, wq_ref, wk_ref, wv_ref, wg_ref, cq_ref, sq_ref,
             ck_ref, sk_ref, q_ref, k_ref, v_ref, g_ref, kc_ref, vc_ref):
    hb = hs_ref[:]
    q = jnp.dot(hb, wq_ref[:], preferred_element_type=jnp.float32)
    q_ref[:] = _rope2d(q, cq_ref[:], sq_ref[:])
    k = jnp.dot(hb, wk_ref[:], preferred_element_type=jnp.float32)
    kr = _rope2d(k, ck_ref[:], sk_ref[:])
    k_ref[:] = kr
    v = jnp.dot(hb, wv_ref[:], preferred_element_type=jnp.float32)
    v_ref[:] = v
    g_ref[:] = jax.nn.sigmoid(
        jnp.dot(hb, wg_ref[:], preferred_element_type=jnp.float32))
    nc = RB // BS
    ci = jax.lax.broadcasted_iota(jnp.int32, (nc, RB), 0)
    ri = jax.lax.broadcasted_iota(jnp.int32, (nc, RB), 1)
    P = jnp.where(ri // BS == ci, 1.0 / BS, 0.0).astype(jnp.float32)
    kc_ref[0] = jnp.dot(P, kr, preferred_element_type=jnp.float32)
    vc_ref[0] = jnp.dot(P, v, preferred_element_type=jnp.float32)


def _softmax_rows(s):
    m = jnp.max(s, axis=-1, keepdims=True)
    e = jnp.exp(s - m)
    return e / jnp.sum(e, axis=-1, keepdims=True)


def _k2_body(nqb, nc, sblk, q_ref, k_ref, v_ref, kc_ref, vc_ref, g_ref,
             wo_ref, e_ref, out_ref):
    qi = pl.program_id(0)
    T = nc * BS
    scale = D ** -0.5

    @pl.when(pl.program_id(1) == 0)
    def _():
        out_ref[:] = jnp.zeros_like(out_ref)

    trow = qi * TQ + jax.lax.broadcasted_iota(jnp.int32, (TQ, 1), 0)
    c32 = jax.lax.broadcasted_iota(jnp.int32, (TQ, nc), 1)
    vis = trow >= (c32 + 1) * BS - 1
    selectable = c32 * BS <= trow
    cur = c32 == trow // BS
    cols = jax.lax.broadcasted_iota(jnp.int32, (TQ, T), 1)
    causal = trow >= cols
    win = (trow - cols) < WIN

    kh = k_ref[0]
    vh = v_ref[0]
    kch = kc_ref[0]
    vch = vc_ref[0]

    nt = (((1,), (1,)), ((), ()))
    imp = jnp.zeros((TQ, nc), jnp.float32)
    o_cmp = []
    for g in range(G):
        qt = q_ref[0, g]
        sc = jax.lax.dot_general(qt, kch, nt,
                                 preferred_element_type=jnp.float32) * scale
        p = _softmax_rows(jnp.where(vis, sc, NEG))
        p = jnp.where(vis, p, 0.0)
        imp = imp + p
        o_cmp.append(jnp.dot(p, vch, preferred_element_type=jnp.float32))
    impv = jnp.where(selectable, imp + jnp.where(cur, 1e9, 0.0), NEG)
    a = impv[:, None, :]
    b = impv[:, :, None]
    cpi = jax.lax.broadcasted_iota(jnp.int32, (1, nc, nc), 2)
    ci = jax.lax.broadcasted_iota(jnp.int32, (1, nc, nc), 1)
    gt = (a > b).astype(jnp.float32)
    eq = ((a == b) & (cpi < ci)).astype(jnp.float32)
    rank = jnp.sum(gt + eq, axis=2)
    sel = (rank < sblk).astype(jnp.float32)
    sel64 = jnp.dot(sel, e_ref[:], preferred_element_type=jnp.float32)
    slc_mask = (sel64 > 0.5) & causal
    swa_mask = causal & win

    for g in range(G):
        qt = q_ref[0, g]
        s = jax.lax.dot_general(qt, kh, nt,
                                preferred_element_type=jnp.float32) * scale
        p_slc = _softmax_rows(jnp.where(slc_mask, s, NEG))
        o_slc = jnp.dot(p_slc, vh, preferred_element_type=jnp.float32)
        p_swa = _softmax_rows(jnp.where(swa_mask, s, NEG))
        o_swa = jnp.dot(p_swa, vh, preferred_element_type=jnp.float32)
        gb = g_ref[0, g]
        oh = (gb[:, 0:1] * o_cmp[g] + gb[:, 1:2] * o_slc
              + gb[:, 2:3] * o_swa)
        out_ref[:] += jnp.dot(oh, wo_ref[0, g],
                              preferred_element_type=jnp.float32)


def kernel(hidden_states, Wq, Wk, Wv, Wg, Wo):
    B, T, HID = hidden_states.shape
    hs = hidden_states.reshape(T, HID)
    nc = T // BS
    sblk = min(SBLK, nc)
    nrb = T // RB
    nqb = T // TQ

    inv = 1.0 / (THETA ** (jnp.arange(32, dtype=jnp.float32) / 32.0))
    fr = jnp.outer(jnp.arange(T, dtype=jnp.float32), inv)
    cosT, sinT = jnp.cos(fr), jnp.sin(fr)
    cq = jnp.tile(jnp.concatenate([cosT, cosT], axis=1), (1, H))
    sq = jnp.tile(jnp.concatenate([-sinT, sinT], axis=1), (1, H))
    ck = jnp.tile(jnp.concatenate([cosT, cosT], axis=1), (1, HKV))
    sk = jnp.tile(jnp.concatenate([-sinT, sinT], axis=1), (1, HKV))

    wqT, wkT, wvT, wgT = Wq.T, Wk.T, Wv.T, Wg.T

    full = lambda shape: pl.BlockSpec(shape, lambda i: tuple(0 for _ in shape))
    q2d, k2d, v2d, g2d, kc3, vc3 = pl.pallas_call(
        _k1_body,
        grid=(nrb,),
        in_specs=[
            pl.BlockSpec((RB, HID), lambda i: (i, 0)),
            full((HID, H * D)), full((HID, HKV * D)), full((HID, HKV * D)),
            full((HID, H * 3)),
            pl.BlockSpec((RB, H * D), lambda i: (i, 0)),
            pl.BlockSpec((RB, H * D), lambda i: (i, 0)),
            pl.BlockSpec((RB, HKV * D), lambda i: (i, 0)),
            pl.BlockSpec((RB, HKV * D), lambda i: (i, 0)),
        ],
        out_specs=[
            pl.BlockSpec((RB, H * D), lambda i: (i, 0)),
            pl.BlockSpec((RB, HKV * D), lambda i: (i, 0)),
            pl.BlockSpec((RB, HKV * D), lambda i: (i, 0)),
            pl.BlockSpec((RB, H * 3), lambda i: (i, 0)),
            pl.BlockSpec((1, RB // BS, HKV * D), lambda i: (i, 0, 0)),
            pl.BlockSpec((1, RB // BS, HKV * D), lambda i: (i, 0, 0)),
        ],
        out_shape=[
            jax.ShapeDtypeStruct((T, H * D), jnp.float32),
            jax.ShapeDtypeStruct((T, HKV * D), jnp.float32),
            jax.ShapeDtypeStruct((T, HKV * D), jnp.float32),
            jax.ShapeDtypeStruct((T, H * 3), jnp.float32),
            jax.ShapeDtypeStruct((nrb, RB // BS, HKV * D), jnp.float32),
            jax.ShapeDtypeStruct((nrb, RB // BS, HKV * D), jnp.float32),
        ],
    )(hs, wqT, wkT, wvT, wgT, cq, sq, ck, sk)

    q4 = q2d.reshape(T, HKV, G, D).transpose(1, 2, 0, 3)
    k4 = k2d.reshape(T, HKV, D).transpose(1, 0, 2)
    v4 = v2d.reshape(T, HKV, D).transpose(1, 0, 2)
    kc4 = kc3.reshape(nc, HKV, D).transpose(1, 0, 2)
    vc4 = vc3.reshape(nc, HKV, D).transpose(1, 0, 2)
    g4 = g2d.reshape(T, HKV, G, 3).transpose(1, 2, 0, 3)
    wo4 = Wo.T.reshape(HKV, G, D, HID)
    eci = jax.lax.broadcasted_iota(jnp.int32, (nc, T), 0)
    eti = jax.lax.broadcasted_iota(jnp.int32, (nc, T), 1)
    emat = (eti // BS == eci).astype(jnp.float32)

    out = pl.pallas_call(
        functools.partial(_k2_body, nqb, nc, sblk),
        grid=(nqb, HKV),
        in_specs=[
            pl.BlockSpec((1, G, TQ, D), lambda i, h: (h, 0, i, 0)),
            pl.BlockSpec((1, T, D), lambda i, h: (h, 0, 0)),
            pl.BlockSpec((1, T, D), lambda i, h: (h, 0, 0)),
            pl.BlockSpec((1, nc, D), lambda i, h: (h, 0, 0)),
            pl.BlockSpec((1, nc, D), lambda i, h: (h, 0, 0)),
            pl.BlockSpec((1, G, TQ, 3), lambda i, h: (h, 0, i, 0)),
            pl.BlockSpec((1, G, D, HID), lambda i, h: (h, 0, 0, 0)),
            pl.BlockSpec((nc, T), lambda i, h: (0, 0)),
        ],
        out_specs=pl.BlockSpec((TQ, HID), lambda i, h: (i, 0)),
        out_shape=jax.ShapeDtypeStruct((T, HID), jnp.float32),
    )(q4, k4, v4, kc4, vc4, g4, wo4, emat)

    return out.reshape(B, T, HID)
